# baseline (speedup 1.0000x reference)
import functools

import jax
import jax.numpy as jnp
import numpy as np
from jax import lax
from jax.experimental import pallas as pl
from jax.experimental.pallas import tpu as pltpu
from jax.experimental.pallas import tpu_sc as plsc

T, D, H, E = 2048, 1024, 16, 8
CAP = 512
DK = D // H
EPS = 1e-5
RB = 512
BLK = 64
NP = T * 2 + E * BLK
NBLK = NP // BLK
RC = 256

F32 = jnp.float32
BF16 = jnp.bfloat16
I32 = jnp.int32
U32 = jnp.uint32
DH = D // 2


def _pack_bf16(x):
    xi = lax.bitcast_convert_type(x, U32)
    pk = (xi[:, DH:] & jnp.uint32(0xFFFF0000)) | (xi[:, :DH] >> 16)
    return lax.bitcast_convert_type(pk, F32)


def _unpack_bf16(p):
    pi = lax.bitcast_convert_type(p, U32)
    lo = lax.bitcast_convert_type(pi << 16, F32)
    hi = lax.bitcast_convert_type(pi & jnp.uint32(0xFFFF0000), F32)
    return jnp.concatenate([lo, hi], axis=1)

INTERP = False


def _ln(x, g, b):
    m = jnp.mean(x, axis=-1, keepdims=True)
    v = jnp.mean((x - m) ** 2, axis=-1, keepdims=True)
    return (x - m) / jnp.sqrt(v + EPS) * g + b


def _qkv_body(x_ref, g_ref, b_ref, wq_ref, wk_ref, wv_ref, q_ref, k_ref, v_ref):
    xn = _ln(x_ref[...], g_ref[...], b_ref[...])
    q = jnp.dot(xn, wq_ref[...], preferred_element_type=F32)
    k = jnp.dot(xn, wk_ref[...], preferred_element_type=F32)
    v = jnp.dot(xn, wv_ref[...], preferred_element_type=F32)
    for h in range(H):
        q_ref[h] = q[:, h * DK:(h + 1) * DK]
        k_ref[h] = k[:, h * DK:(h + 1) * DK]
        v_ref[h] = v[:, h * DK:(h + 1) * DK]


def _attn_body(q_ref, k_ref, v_ref, o_ref):
    q = q_ref[0]
    k = k_ref[0]
    s = lax.dot_general(q, k, (((1,), (1,)), ((), ())),
                        preferred_element_type=F32) * (1.0 / np.sqrt(DK))
    s = s - jnp.max(s, axis=-1, keepdims=True)
    p = jnp.exp(s)
    p = p / jnp.sum(p, axis=-1, keepdims=True)
    o_ref[0] = jnp.dot(p, v_ref[0], preferred_element_type=F32)


def _post_body(a_ref, x_ref, wo_ref, wg_ref, g_ref, b_ref,
               x2_ref, xn2_ref, gs_ref):
    a = jnp.concatenate([a_ref[h] for h in range(H)], axis=1)
    x2 = x_ref[...] + jnp.dot(a, wo_ref[...], preferred_element_type=F32)
    xn2 = _ln(x2, g_ref[...], b_ref[...])
    x2_ref[...] = x2
    xn2_ref[...] = _pack_bf16(xn2)
    gs_ref[...] = jnp.dot(xn2, wg_ref[...], preferred_element_type=F32)


def _route_body(g_ref, rt_ref, be_ref, mask_s, rank_s):
    g = g_ref[...]
    col = lax.broadcasted_iota(I32, (T, E), 1)
    m1 = jnp.max(g, axis=1, keepdims=True)
    i1 = jnp.min(jnp.where(g >= m1, col, E), axis=1, keepdims=True)
    gm = jnp.where(col == i1, -jnp.inf, g)
    m2 = jnp.max(gm, axis=1, keepdims=True)
    i2 = jnp.min(jnp.where(gm >= m2, col, E), axis=1, keepdims=True)
    e2 = jnp.exp(m2 - m1)
    w1 = 1.0 / (1.0 + e2)
    w2 = e2 * w1
    mask_s[...] = ((col == i1) | (col == i2)).astype(F32)

    tri = (lax.broadcasted_iota(I32, (RC, RC), 0)
           > lax.broadcasted_iota(I32, (RC, RC), 1)).astype(F32)

    def body(c, carry):
        rows = mask_s[pl.ds(c * RC, RC), :]
        rank_s[pl.ds(c * RC, RC), :] = jnp.dot(
            tri, rows, preferred_element_type=F32) + carry
        return carry + jnp.sum(rows, axis=0, keepdims=True)

    cnt = lax.fori_loop(0, T // RC, body, jnp.zeros((1, E), F32))
    pc = ((cnt.astype(I32) + (BLK - 1)) // BLK * BLK).astype(F32)
    upper = (lax.broadcasted_iota(I32, (E, E), 0)
             < lax.broadcasted_iota(I32, (E, E), 1)).astype(F32)
    poff = jnp.dot(pc, upper, preferred_element_type=F32)
    pend = poff + pc
    rank = rank_s[...]
    pos_all = poff + rank
    pos1 = jnp.sum(jnp.where(col == i1, pos_all, 0.0), axis=1, keepdims=True)
    pos2 = jnp.sum(jnp.where(col == i2, pos_all, 0.0), axis=1, keepdims=True)
    rt = jnp.where(col == 0, pos1,
         jnp.where(col == 1, pos2,
         jnp.where(col == 2, w1, w2)))
    rt_ref[...] = rt

    bidx = lax.broadcasted_iota(I32, (NBLK, E), 0).astype(F32) * BLK
    nfull = jnp.sum((pend <= bidx).astype(F32), axis=1, keepdims=True)
    be_ref[...] = jnp.minimum(nfull, E - 1) + jnp.zeros((NBLK, E), F32)


def _expert_body(be_ref, x_ref, w1_ref, w2_ref, y_ref):
    x = _unpack_bf16(x_ref[...])
    h = jnp.maximum(jnp.dot(x, w1_ref[0], preferred_element_type=F32), 0.0)
    y_ref[...] = _pack_bf16(jnp.dot(h, w2_ref[0], preferred_element_type=F32))


def _final_body(x2_ref, g0_ref, g1_ref, rt_ref, o_ref):
    w1 = rt_ref[:, 2:3]
    w2 = rt_ref[:, 3:4]
    o_ref[...] = (x2_ref[...] + w1 * _unpack_bf16(g0_ref[...])
                  + w2 * _unpack_bf16(g1_ref[...]))


def _qkv_call(x2d, g1, b1, Wq, Wk, Wv):
    grid = (T // RB,)
    row = pl.BlockSpec((RB, D), lambda r: (r, 0))
    full = pl.BlockSpec((D, D), lambda r: (0, 0))
    vec = pl.BlockSpec((1, D), lambda r: (0, 0))
    hrow = pl.BlockSpec((H, RB, DK), lambda r: (0, r, 0))
    return pl.pallas_call(
        _qkv_body, grid=grid,
        in_specs=[row, vec, vec, full, full, full],
        out_specs=[hrow, hrow, hrow],
        out_shape=[jax.ShapeDtypeStruct((H, T, DK), F32)] * 3,
        interpret=INTERP,
    )(x2d, g1, b1, Wq, Wk, Wv)


def _attn_call(Q3, K3, V3):
    grid = (H, T // RB)
    qspec = pl.BlockSpec((1, RB, DK), lambda h, r: (h, r, 0))
    kspec = pl.BlockSpec((1, T, DK), lambda h, r: (h, 0, 0))
    return pl.pallas_call(
        _attn_body, grid=grid,
        in_specs=[qspec, kspec, kspec],
        out_specs=qspec,
        out_shape=jax.ShapeDtypeStruct((H, T, DK), F32),
        interpret=INTERP,
    )(Q3, K3, V3)


def _post_call(A, x2d, Wo, Wg, g2, b2):
    grid = (T // RB,)
    row = pl.BlockSpec((RB, D), lambda r: (r, 0))
    return pl.pallas_call(
        _post_body, grid=grid,
        in_specs=[pl.BlockSpec((H, RB, DK), lambda r: (0, r, 0)), row,
                  pl.BlockSpec((D, D), lambda r: (0, 0)),
                  pl.BlockSpec((D, E), lambda r: (0, 0)),
                  pl.BlockSpec((1, D), lambda r: (0, 0)),
                  pl.BlockSpec((1, D), lambda r: (0, 0))],
        out_specs=[row, pl.BlockSpec((RB, DH), lambda r: (r, 0)),
                   pl.BlockSpec((RB, E), lambda r: (r, 0))],
        out_shape=[jax.ShapeDtypeStruct((T, D), F32),
                   jax.ShapeDtypeStruct((T, DH), F32),
                   jax.ShapeDtypeStruct((T, E), F32)],
        interpret=INTERP,
    )(A, x2d, Wo, Wg, g2, b2)


def _route_call(gs):
    return pl.pallas_call(
        _route_body,
        in_specs=[pl.BlockSpec((T, E), lambda: (0, 0))],
        out_specs=[pl.BlockSpec((T, E), lambda: (0, 0)),
                   pl.BlockSpec((NBLK, E), lambda: (0, 0))],
        out_shape=[jax.ShapeDtypeStruct((T, E), F32),
                   jax.ShapeDtypeStruct((NBLK, E), F32)],
        scratch_shapes=[pltpu.VMEM((T, E), F32), pltpu.VMEM((T, E), F32)],
        interpret=INTERP,
    )(gs)


def _expert_call(be, Xs, W1, W2):
    nblk, npad = be.shape[0], Xs.shape[0]
    grid_spec = pltpu.PrefetchScalarGridSpec(
        num_scalar_prefetch=1,
        grid=(nblk,),
        in_specs=[pl.BlockSpec((BLK, DH), lambda b, be: (b, 0)),
                  pl.BlockSpec((1, D, CAP), lambda b, be: (be[b], 0, 0)),
                  pl.BlockSpec((1, CAP, D), lambda b, be: (be[b], 0, 0))],
        out_specs=pl.BlockSpec((BLK, DH), lambda b, be: (b, 0)),
    )
    return pl.pallas_call(
        _expert_body, grid_spec=grid_spec,
        out_shape=jax.ShapeDtypeStruct((npad, DH), F32),
        interpret=INTERP,
    )(be, Xs, W1, W2)


def _final_call(x2, G, rt):
    grid = (T // RB,)
    row = pl.BlockSpec((RB, D), lambda r: (r, 0))
    return pl.pallas_call(
        _final_body, grid=grid,
        in_specs=[row,
                  pl.BlockSpec((RB, DH), lambda r: (r, 0)),
                  pl.BlockSpec((RB, DH), lambda r: (T // RB + r, 0)),
                  pl.BlockSpec((RB, E), lambda r: (r, 0))],
        out_specs=row,
        out_shape=jax.ShapeDtypeStruct((T, D), F32),
        interpret=INTERP,
    )(x2, G, G, rt)


_SC_NC = 2
_SC_NS = 16
_NW = _SC_NC * _SC_NS


def _sc_mesh():
    return plsc.VectorSubcoreMesh(core_axis_name="c", subcore_axis_name="s",
                                  num_cores=_SC_NC, num_subcores=_SC_NS)


def _sc_wid():
    return lax.axis_index("s") * _SC_NC + lax.axis_index("c")


def _gather_ring(table_hbm, out_hbm, idx_v, rows_v, gsem, osem, base, CH, nc,
                 nbuf):
    def start_gather(i, b):
        return pltpu.async_copy(
            table_hbm.at[idx_v.at[pl.ds(i * CH, CH)]], rows_v.at[b], gsem[b])

    hg = [start_gather(i, i) for i in range(nbuf)]
    ho = [None] * nbuf
    for i in range(nc):
        b = i % nbuf
        hg[b].wait()
        ho[b] = pltpu.async_copy(
            rows_v.at[b], out_hbm.at[pl.ds(base + i * CH, CH)], osem[b])
        if i + nbuf < nc:
            ho[b].wait()
            hg[b] = start_gather(i + nbuf, b)
    for i in range(max(nc - nbuf, 0), nc):
        ho[i % nbuf].wait()


@functools.lru_cache(maxsize=None)
def _get_dispatch():
    @functools.partial(
        pl.kernel,
        out_type=jax.ShapeDtypeStruct((NP,), I32),
        mesh=_sc_mesh(),
        scratch_types=[pltpu.VMEM((NP,), I32), pltpu.VMEM((T,), I32)],
        compiler_params=pltpu.CompilerParams(needs_layout_passes=False),
    )
    def _dispatch_kernel(pos1_hbm, pos2_hbm, zeros_hbm, src_hbm, src_v, pos_v):
        @pl.when(_sc_wid() == 0)
        def _():
            pltpu.sync_copy(zeros_hbm, src_v)
            ids = lax.iota(I32, 16)

            def scat(pos_hbm):
                pltpu.sync_copy(pos_hbm, pos_v)

                def body(i, _):
                    idx = pos_v[pl.ds(i * 16, 16)]
                    plsc.store_scatter(src_v, [idx], i * 16 + ids)
                    return 0
                lax.fori_loop(0, T // 16, body, 0)

            scat(pos1_hbm)
            scat(pos2_hbm)
            pltpu.sync_copy(src_v, src_hbm)

    return _dispatch_kernel


@functools.lru_cache(maxsize=None)
def _make_gather(N, DW=DH, CH=16, nbuf=4):
    n_per = N // _NW
    nc = n_per // CH

    @functools.partial(
        pl.kernel,
        out_type=jax.ShapeDtypeStruct((N, DW), F32),
        mesh=_sc_mesh(),
        scratch_types=[pltpu.VMEM((n_per,), I32),
                       pltpu.VMEM((nbuf, CH, DW), F32)]
                      + [pltpu.SemaphoreType.DMA] * (2 * nbuf),
    )
    def gk(table_hbm, idx_hbm, out_hbm, idx_v, rows_v, *sems):
        base = _sc_wid() * n_per
        pltpu.sync_copy(idx_hbm.at[pl.ds(base, n_per)], idx_v)
        _gather_ring(table_hbm, out_hbm, idx_v, rows_v,
                     sems[:nbuf], sems[nbuf:], base, CH, nc, nbuf)

    return gk


def _dispatch_src(pos1, pos2):
    return _get_dispatch()(pos1, pos2, jnp.zeros((NP,), I32))


def _gather_rows(table, idx):
    return _make_gather(idx.shape[0])(table, idx)


def kernel(x, Wq, Wk, Wv, Wo, Wg, W1, W2, gamma1, beta1, gamma2, beta2):
    x2d = x.reshape(T, D)
    g1 = gamma1.reshape(1, D)
    b1 = beta1.reshape(1, D)
    g2 = gamma2.reshape(1, D)
    b2 = beta2.reshape(1, D)

    Q3, K3, V3 = _qkv_call(x2d, g1, b1, Wq, Wk, Wv)
    A3 = _attn_call(Q3, K3, V3)
    x2, xn2, gs = _post_call(A3, x2d, Wo, Wg, g2, b2)
    rt, be = _route_call(gs)

    pos1 = rt[:, 0].astype(I32)
    pos2 = rt[:, 1].astype(I32)
    beids = be[:, 0].astype(I32)

    src = _dispatch_src(pos1, pos2)
    nph = NP // 2
    XsA = _make_gather(nph, CH=8, nbuf=6)(xn2, src[:nph])
    XsB = _make_gather(nph, CH=8, nbuf=6)(xn2, src[nph:])
    YsA = _expert_call(beids[:NBLK // 2], XsA, W1, W2)
    YsB = _expert_call(beids[NBLK // 2:], XsB, W1, W2)
    Ys = jnp.concatenate([YsA, YsB], axis=0)
    pos12 = jnp.concatenate([pos1, pos2])
    G = _gather_rows(Ys, pos12)
    out = _final_call(x2, G, rt)
    return out.reshape(1, T, D)


# speedup vs baseline: 1.0081x; 1.0081x over previous
import functools

import jax
import jax.numpy as jnp
import numpy as np
from jax import lax
from jax.experimental import pallas as pl
from jax.experimental.pallas import tpu as pltpu
from jax.experimental.pallas import tpu_sc as plsc

T, D, H, E = 2048, 1024, 16, 8
CAP = 512
DK = D // H
EPS = 1e-5
RB = 512
BLK = 64
NP = T * 2 + E * BLK
NBLK = NP // BLK
RC = 256

F32 = jnp.float32
BF16 = jnp.bfloat16
I32 = jnp.int32
U32 = jnp.uint32
DH = D // 2


def _pack_bf16(x):
    xi = lax.bitcast_convert_type(x, U32)
    pk = (xi[:, DH:] & jnp.uint32(0xFFFF0000)) | (xi[:, :DH] >> 16)
    return lax.bitcast_convert_type(pk, F32)


def _unpack_bf16(p):
    pi = lax.bitcast_convert_type(p, U32)
    lo = lax.bitcast_convert_type(pi << 16, F32)
    hi = lax.bitcast_convert_type(pi & jnp.uint32(0xFFFF0000), F32)
    return jnp.concatenate([lo, hi], axis=1)

INTERP = False


def _ln(x, g, b):
    m = jnp.mean(x, axis=-1, keepdims=True)
    v = jnp.mean((x - m) ** 2, axis=-1, keepdims=True)
    return (x - m) / jnp.sqrt(v + EPS) * g + b


def _qkv_body(x_ref, g_ref, b_ref, wq_ref, wk_ref, wv_ref, q_ref, k_ref, v_ref):
    xn = _ln(x_ref[...], g_ref[...], b_ref[...])
    q = jnp.dot(xn, wq_ref[...], preferred_element_type=F32)
    k = jnp.dot(xn, wk_ref[...], preferred_element_type=F32)
    v = jnp.dot(xn, wv_ref[...], preferred_element_type=F32)
    for h in range(H):
        q_ref[h] = q[:, h * DK:(h + 1) * DK]
        k_ref[h] = k[:, h * DK:(h + 1) * DK]
        v_ref[h] = v[:, h * DK:(h + 1) * DK]


def _attn_body(q_ref, k_ref, v_ref, o_ref):
    q = q_ref[0]
    k = k_ref[0]
    s = lax.dot_general(q, k, (((1,), (1,)), ((), ())),
                        preferred_element_type=F32) * (1.0 / np.sqrt(DK))
    s = s - jnp.max(s, axis=-1, keepdims=True)
    p = jnp.exp(s)
    p = p / jnp.sum(p, axis=-1, keepdims=True)
    o_ref[0] = jnp.dot(p, v_ref[0], preferred_element_type=F32)


def _post_body(a_ref, x_ref, wo_ref, wg_ref, g_ref, b_ref,
               x2_ref, xn2_ref, gs_ref):
    a = jnp.concatenate([a_ref[h] for h in range(H)], axis=1)
    x2 = x_ref[...] + jnp.dot(a, wo_ref[...], preferred_element_type=F32)
    xn2 = _ln(x2, g_ref[...], b_ref[...])
    x2_ref[...] = x2
    xn2_ref[...] = _pack_bf16(xn2)
    gs_ref[...] = jnp.dot(xn2, wg_ref[...], preferred_element_type=F32)


def _route_body(g_ref, rt_ref, be_ref, mask_s, rank_s):
    g = g_ref[...]
    col = lax.broadcasted_iota(I32, (T, E), 1)
    m1 = jnp.max(g, axis=1, keepdims=True)
    i1 = jnp.min(jnp.where(g >= m1, col, E), axis=1, keepdims=True)
    gm = jnp.where(col == i1, -jnp.inf, g)
    m2 = jnp.max(gm, axis=1, keepdims=True)
    i2 = jnp.min(jnp.where(gm >= m2, col, E), axis=1, keepdims=True)
    e2 = jnp.exp(m2 - m1)
    w1 = 1.0 / (1.0 + e2)
    w2 = e2 * w1
    mask_s[...] = ((col == i1) | (col == i2)).astype(F32)

    tri = (lax.broadcasted_iota(I32, (RC, RC), 0)
           > lax.broadcasted_iota(I32, (RC, RC), 1)).astype(F32)

    def body(c, carry):
        rows = mask_s[pl.ds(c * RC, RC), :]
        rank_s[pl.ds(c * RC, RC), :] = jnp.dot(
            tri, rows, preferred_element_type=F32) + carry
        return carry + jnp.sum(rows, axis=0, keepdims=True)

    cnt = lax.fori_loop(0, T // RC, body, jnp.zeros((1, E), F32))
    pc = ((cnt.astype(I32) + (BLK - 1)) // BLK * BLK).astype(F32)
    upper = (lax.broadcasted_iota(I32, (E, E), 0)
             < lax.broadcasted_iota(I32, (E, E), 1)).astype(F32)
    poff = jnp.dot(pc, upper, preferred_element_type=F32)
    pend = poff + pc
    rank = rank_s[...]
    pos_all = poff + rank
    pos1 = jnp.sum(jnp.where(col == i1, pos_all, 0.0), axis=1, keepdims=True)
    pos2 = jnp.sum(jnp.where(col == i2, pos_all, 0.0), axis=1, keepdims=True)
    rt = jnp.where(col == 0, pos1,
         jnp.where(col == 1, pos2,
         jnp.where(col == 2, w1, w2)))
    rt_ref[...] = rt

    bidx = lax.broadcasted_iota(I32, (NBLK, E), 0).astype(F32) * BLK
    nfull = jnp.sum((pend <= bidx).astype(F32), axis=1, keepdims=True)
    be_ref[...] = jnp.minimum(nfull, E - 1) + jnp.zeros((NBLK, E), F32)


def _expert_body(be_ref, x_ref, w1_ref, w2_ref, y_ref):
    x = _unpack_bf16(x_ref[...])
    h = jnp.maximum(jnp.dot(x, w1_ref[0], preferred_element_type=F32), 0.0)
    y_ref[...] = _pack_bf16(jnp.dot(h, w2_ref[0], preferred_element_type=F32))


def _final_body(x2_ref, g0_ref, g1_ref, rt_ref, o_ref):
    w1 = rt_ref[:, 2:3]
    w2 = rt_ref[:, 3:4]
    o_ref[...] = (x2_ref[...] + w1 * _unpack_bf16(g0_ref[...])
                  + w2 * _unpack_bf16(g1_ref[...]))


def _qkv_call(x2d, g1, b1, Wq, Wk, Wv):
    grid = (T // RB,)
    row = pl.BlockSpec((RB, D), lambda r: (r, 0))
    full = pl.BlockSpec((D, D), lambda r: (0, 0))
    vec = pl.BlockSpec((1, D), lambda r: (0, 0))
    hrow = pl.BlockSpec((H, RB, DK), lambda r: (0, r, 0))
    return pl.pallas_call(
        _qkv_body, grid=grid,
        in_specs=[row, vec, vec, full, full, full],
        out_specs=[hrow, hrow, hrow],
        out_shape=[jax.ShapeDtypeStruct((H, T, DK), F32)] * 3,
        interpret=INTERP,
    )(x2d, g1, b1, Wq, Wk, Wv)


def _attn_call(Q3, K3, V3):
    grid = (H, T // RB)
    qspec = pl.BlockSpec((1, RB, DK), lambda h, r: (h, r, 0))
    kspec = pl.BlockSpec((1, T, DK), lambda h, r: (h, 0, 0))
    return pl.pallas_call(
        _attn_body, grid=grid,
        in_specs=[qspec, kspec, kspec],
        out_specs=qspec,
        out_shape=jax.ShapeDtypeStruct((H, T, DK), F32),
        interpret=INTERP,
    )(Q3, K3, V3)


def _post_call(A, x2d, Wo, Wg, g2, b2):
    grid = (T // RB,)
    row = pl.BlockSpec((RB, D), lambda r: (r, 0))
    return pl.pallas_call(
        _post_body, grid=grid,
        in_specs=[pl.BlockSpec((H, RB, DK), lambda r: (0, r, 0)), row,
                  pl.BlockSpec((D, D), lambda r: (0, 0)),
                  pl.BlockSpec((D, E), lambda r: (0, 0)),
                  pl.BlockSpec((1, D), lambda r: (0, 0)),
                  pl.BlockSpec((1, D), lambda r: (0, 0))],
        out_specs=[row, pl.BlockSpec((RB, DH), lambda r: (r, 0)),
                   pl.BlockSpec((RB, E), lambda r: (r, 0))],
        out_shape=[jax.ShapeDtypeStruct((T, D), F32),
                   jax.ShapeDtypeStruct((T, DH), F32),
                   jax.ShapeDtypeStruct((T, E), F32)],
        interpret=INTERP,
    )(A, x2d, Wo, Wg, g2, b2)


def _route_call(gs):
    return pl.pallas_call(
        _route_body,
        in_specs=[pl.BlockSpec((T, E), lambda: (0, 0))],
        out_specs=[pl.BlockSpec((T, E), lambda: (0, 0)),
                   pl.BlockSpec((NBLK, E), lambda: (0, 0))],
        out_shape=[jax.ShapeDtypeStruct((T, E), F32),
                   jax.ShapeDtypeStruct((NBLK, E), F32)],
        scratch_shapes=[pltpu.VMEM((T, E), F32), pltpu.VMEM((T, E), F32)],
        interpret=INTERP,
    )(gs)


def _expert_call(be, Xs, W1, W2):
    nblk, npad = be.shape[0], Xs.shape[0]
    grid_spec = pltpu.PrefetchScalarGridSpec(
        num_scalar_prefetch=1,
        grid=(nblk,),
        in_specs=[pl.BlockSpec((BLK, DH), lambda b, be: (b, 0)),
                  pl.BlockSpec((1, D, CAP), lambda b, be: (be[b], 0, 0)),
                  pl.BlockSpec((1, CAP, D), lambda b, be: (be[b], 0, 0))],
        out_specs=pl.BlockSpec((BLK, DH), lambda b, be: (b, 0)),
    )
    return pl.pallas_call(
        _expert_body, grid_spec=grid_spec,
        out_shape=jax.ShapeDtypeStruct((npad, DH), F32),
        interpret=INTERP,
    )(be, Xs, W1, W2)


def _final_call(x2, G, rt):
    grid = (T // RB,)
    row = pl.BlockSpec((RB, D), lambda r: (r, 0))
    return pl.pallas_call(
        _final_body, grid=grid,
        in_specs=[row,
                  pl.BlockSpec((RB, DH), lambda r: (r, 0)),
                  pl.BlockSpec((RB, DH), lambda r: (T // RB + r, 0)),
                  pl.BlockSpec((RB, E), lambda r: (r, 0))],
        out_specs=row,
        out_shape=jax.ShapeDtypeStruct((T, D), F32),
        interpret=INTERP,
    )(x2, G, G, rt)


_SC_NC = 2
_SC_NS = 16
_NW = _SC_NC * _SC_NS


def _sc_mesh():
    return plsc.VectorSubcoreMesh(core_axis_name="c", subcore_axis_name="s",
                                  num_cores=_SC_NC, num_subcores=_SC_NS)


def _sc_wid():
    return lax.axis_index("s") * _SC_NC + lax.axis_index("c")


def _gather_ring(table_hbm, out_hbm, idx_v, rows_v, gsem, osem, base, CH, nc,
                 nbuf):
    def start_gather(i, b):
        return pltpu.async_copy(
            table_hbm.at[idx_v.at[pl.ds(i * CH, CH)]], rows_v.at[b], gsem[b])

    hg = [start_gather(i, i) for i in range(nbuf)]
    ho = [None] * nbuf
    for i in range(nc):
        b = i % nbuf
        hg[b].wait()
        ho[b] = pltpu.async_copy(
            rows_v.at[b], out_hbm.at[pl.ds(base + i * CH, CH)], osem[b])
        if i + nbuf < nc:
            ho[b].wait()
            hg[b] = start_gather(i + nbuf, b)
    for i in range(max(nc - nbuf, 0), nc):
        ho[i % nbuf].wait()


@functools.lru_cache(maxsize=None)
def _get_dispatch():
    @functools.partial(
        pl.kernel,
        out_type=jax.ShapeDtypeStruct((NP,), I32),
        mesh=_sc_mesh(),
        scratch_types=[pltpu.VMEM((NP,), I32), pltpu.VMEM((T,), I32)],
        compiler_params=pltpu.CompilerParams(needs_layout_passes=False),
    )
    def _dispatch_kernel(pos1_hbm, pos2_hbm, zeros_hbm, src_hbm, src_v, pos_v):
        @pl.when(_sc_wid() == 0)
        def _():
            pltpu.sync_copy(zeros_hbm, src_v)
            ids = lax.iota(I32, 16)

            def scat(pos_hbm):
                pltpu.sync_copy(pos_hbm, pos_v)

                def body(i, _):
                    idx = pos_v[pl.ds(i * 16, 16)]
                    plsc.store_scatter(src_v, [idx], i * 16 + ids)
                    return 0
                lax.fori_loop(0, T // 16, body, 0)

            scat(pos1_hbm)
            scat(pos2_hbm)
            pltpu.sync_copy(src_v, src_hbm)

    return _dispatch_kernel


@functools.lru_cache(maxsize=None)
def _make_gather(N, DW=DH, CH=16, nbuf=4):
    n_per = N // _NW
    nc = n_per // CH

    @functools.partial(
        pl.kernel,
        out_type=jax.ShapeDtypeStruct((N, DW), F32),
        mesh=_sc_mesh(),
        scratch_types=[pltpu.VMEM((n_per,), I32),
                       pltpu.VMEM((nbuf, CH, DW), F32)]
                      + [pltpu.SemaphoreType.DMA] * (2 * nbuf),
    )
    def gk(table_hbm, idx_hbm, out_hbm, idx_v, rows_v, *sems):
        base = _sc_wid() * n_per
        pltpu.sync_copy(idx_hbm.at[pl.ds(base, n_per)], idx_v)
        _gather_ring(table_hbm, out_hbm, idx_v, rows_v,
                     sems[:nbuf], sems[nbuf:], base, CH, nc, nbuf)

    return gk


def _dispatch_src(pos1, pos2):
    return _get_dispatch()(pos1, pos2, jnp.zeros((NP,), I32))


def _gather_rows(table, idx):
    return _make_gather(idx.shape[0])(table, idx)


def kernel(x, Wq, Wk, Wv, Wo, Wg, W1, W2, gamma1, beta1, gamma2, beta2):
    x2d = x.reshape(T, D)
    g1 = gamma1.reshape(1, D)
    b1 = beta1.reshape(1, D)
    g2 = gamma2.reshape(1, D)
    b2 = beta2.reshape(1, D)

    Q3, K3, V3 = _qkv_call(x2d, g1, b1, Wq, Wk, Wv)
    A3 = _attn_call(Q3, K3, V3)
    x2, xn2, gs = _post_call(A3, x2d, Wo, Wg, g2, b2)
    rt, be = _route_call(gs)

    pos1 = rt[:, 0].astype(I32)
    pos2 = rt[:, 1].astype(I32)
    beids = be[:, 0].astype(I32)

    src = _dispatch_src(pos1, pos2)
    Xs = _make_gather(NP, CH=8, nbuf=6)(xn2, src)
    Ys = _expert_call(beids, Xs, W1, W2)
    pos12 = jnp.concatenate([pos1, pos2])
    G = _gather_rows(Ys, pos12)
    out = _final_call(x2, G, rt)
    return out.reshape(1, T, D)


# speedup vs baseline: 1.0343x; 1.0260x over previous
import functools

import jax
import jax.numpy as jnp
import numpy as np
from jax import lax
from jax.experimental import pallas as pl
from jax.experimental.pallas import tpu as pltpu
from jax.experimental.pallas import tpu_sc as plsc

T, D, H, E = 2048, 1024, 16, 8
CAP = 512
DK = D // H
EPS = 1e-5
RB = 512
BLK = 64
NP = T * 2 + E * BLK
NBLK = NP // BLK
RC = 256

F32 = jnp.float32
BF16 = jnp.bfloat16
I32 = jnp.int32
U32 = jnp.uint32
DH = D // 2


def _pack_bf16(x):
    xi = lax.bitcast_convert_type(x, U32)
    pk = (xi[:, DH:] & jnp.uint32(0xFFFF0000)) | (xi[:, :DH] >> 16)
    return lax.bitcast_convert_type(pk, F32)


def _unpack_bf16(p):
    pi = lax.bitcast_convert_type(p, U32)
    lo = lax.bitcast_convert_type(pi << 16, F32)
    hi = lax.bitcast_convert_type(pi & jnp.uint32(0xFFFF0000), F32)
    return jnp.concatenate([lo, hi], axis=1)

INTERP = False


def _ln(x, g, b):
    m = jnp.mean(x, axis=-1, keepdims=True)
    v = jnp.mean((x - m) ** 2, axis=-1, keepdims=True)
    return (x - m) / jnp.sqrt(v + EPS) * g + b


def _qkv_body(x_ref, g_ref, b_ref, wq_ref, wk_ref, wv_ref, q_ref, k_ref, v_ref):
    xn = _ln(x_ref[...], g_ref[...], b_ref[...])
    q = jnp.dot(xn, wq_ref[...], preferred_element_type=F32)
    k = jnp.dot(xn, wk_ref[...], preferred_element_type=F32)
    v = jnp.dot(xn, wv_ref[...], preferred_element_type=F32)
    for h in range(H):
        q_ref[h] = q[:, h * DK:(h + 1) * DK]
        k_ref[h] = k[:, h * DK:(h + 1) * DK]
        v_ref[h] = v[:, h * DK:(h + 1) * DK]


def _attn_body(q_ref, k_ref, v_ref, o_ref):
    q = q_ref[0]
    k = k_ref[0]
    s = lax.dot_general(q, k, (((1,), (1,)), ((), ())),
                        preferred_element_type=F32) * (1.0 / np.sqrt(DK))
    s = s - jnp.max(s, axis=-1, keepdims=True)
    p = jnp.exp(s)
    p = p / jnp.sum(p, axis=-1, keepdims=True)
    o_ref[0] = jnp.dot(p, v_ref[0], preferred_element_type=F32)


def _post_body(a_ref, x_ref, wo_ref, wg_ref, g_ref, b_ref,
               x2_ref, xn2_ref, gs_ref):
    a = jnp.concatenate([a_ref[h] for h in range(H)], axis=1)
    x2 = x_ref[...] + jnp.dot(a, wo_ref[...], preferred_element_type=F32)
    xn2 = _ln(x2, g_ref[...], b_ref[...])
    x2_ref[...] = x2
    xn2_ref[...] = _pack_bf16(xn2)
    gs_ref[...] = jnp.dot(xn2, wg_ref[...], preferred_element_type=F32)


def _route_body(g_ref, rt_ref, be_ref, mask_s, rank_s):
    g = g_ref[...]
    col = lax.broadcasted_iota(I32, (T, E), 1)
    m1 = jnp.max(g, axis=1, keepdims=True)
    i1 = jnp.min(jnp.where(g >= m1, col, E), axis=1, keepdims=True)
    gm = jnp.where(col == i1, -jnp.inf, g)
    m2 = jnp.max(gm, axis=1, keepdims=True)
    i2 = jnp.min(jnp.where(gm >= m2, col, E), axis=1, keepdims=True)
    e2 = jnp.exp(m2 - m1)
    w1 = 1.0 / (1.0 + e2)
    w2 = e2 * w1
    mask_s[...] = ((col == i1) | (col == i2)).astype(F32)

    tri = (lax.broadcasted_iota(I32, (RC, RC), 0)
           > lax.broadcasted_iota(I32, (RC, RC), 1)).astype(F32)

    def body(c, carry):
        rows = mask_s[pl.ds(c * RC, RC), :]
        rank_s[pl.ds(c * RC, RC), :] = jnp.dot(
            tri, rows, preferred_element_type=F32) + carry
        return carry + jnp.sum(rows, axis=0, keepdims=True)

    cnt = lax.fori_loop(0, T // RC, body, jnp.zeros((1, E), F32))
    pc = ((cnt.astype(I32) + (BLK - 1)) // BLK * BLK).astype(F32)
    upper = (lax.broadcasted_iota(I32, (E, E), 0)
             < lax.broadcasted_iota(I32, (E, E), 1)).astype(F32)
    poff = jnp.dot(pc, upper, preferred_element_type=F32)
    pend = poff + pc
    rank = rank_s[...]
    pos_all = poff + rank
    pos1 = jnp.sum(jnp.where(col == i1, pos_all, 0.0), axis=1, keepdims=True)
    pos2 = jnp.sum(jnp.where(col == i2, pos_all, 0.0), axis=1, keepdims=True)
    rt = jnp.where(col == 0, pos1,
         jnp.where(col == 1, pos2,
         jnp.where(col == 2, w1, w2)))
    rt_ref[...] = rt

    bidx = lax.broadcasted_iota(I32, (NBLK, E), 0).astype(F32) * BLK
    nfull = jnp.sum((pend <= bidx).astype(F32), axis=1, keepdims=True)
    be_ref[...] = jnp.minimum(nfull, E - 1) + jnp.zeros((NBLK, E), F32)


def _expert_body(be_ref, x_ref, w1_ref, w2_ref, y_ref):
    x = _unpack_bf16(x_ref[...])
    h = jnp.maximum(jnp.dot(x, w1_ref[0], preferred_element_type=F32), 0.0)
    y_ref[...] = _pack_bf16(jnp.dot(h, w2_ref[0], preferred_element_type=F32))


def _final_body(x2_ref, g0_ref, g1_ref, rt_ref, o_ref):
    w1 = rt_ref[:, 2:3]
    w2 = rt_ref[:, 3:4]
    o_ref[...] = (x2_ref[...] + w1 * _unpack_bf16(g0_ref[...])
                  + w2 * _unpack_bf16(g1_ref[...]))


def _qkv_call(x2d, g1, b1, Wq, Wk, Wv):
    grid = (T // RB,)
    row = pl.BlockSpec((RB, D), lambda r: (r, 0))
    full = pl.BlockSpec((D, D), lambda r: (0, 0))
    vec = pl.BlockSpec((1, D), lambda r: (0, 0))
    hrow = pl.BlockSpec((H, RB, DK), lambda r: (0, r, 0))
    return pl.pallas_call(
        _qkv_body, grid=grid,
        in_specs=[row, vec, vec, full, full, full],
        out_specs=[hrow, hrow, hrow],
        out_shape=[jax.ShapeDtypeStruct((H, T, DK), F32)] * 3,
        interpret=INTERP,
    )(x2d, g1, b1, Wq, Wk, Wv)


RA = 1024


def _attn_call(Q3, K3, V3):
    grid = (H, T // RA)
    qspec = pl.BlockSpec((1, RA, DK), lambda h, r: (h, r, 0))
    kspec = pl.BlockSpec((1, T, DK), lambda h, r: (h, 0, 0))
    return pl.pallas_call(
        _attn_body, grid=grid,
        in_specs=[qspec, kspec, kspec],
        out_specs=qspec,
        out_shape=jax.ShapeDtypeStruct((H, T, DK), F32),
        interpret=INTERP,
    )(Q3, K3, V3)


def _post_call(A, x2d, Wo, Wg, g2, b2):
    grid = (T // RB,)
    row = pl.BlockSpec((RB, D), lambda r: (r, 0))
    return pl.pallas_call(
        _post_body, grid=grid,
        in_specs=[pl.BlockSpec((H, RB, DK), lambda r: (0, r, 0)), row,
                  pl.BlockSpec((D, D), lambda r: (0, 0)),
                  pl.BlockSpec((D, E), lambda r: (0, 0)),
                  pl.BlockSpec((1, D), lambda r: (0, 0)),
                  pl.BlockSpec((1, D), lambda r: (0, 0))],
        out_specs=[row, pl.BlockSpec((RB, DH), lambda r: (r, 0)),
                   pl.BlockSpec((RB, E), lambda r: (r, 0))],
        out_shape=[jax.ShapeDtypeStruct((T, D), F32),
                   jax.ShapeDtypeStruct((T, DH), F32),
                   jax.ShapeDtypeStruct((T, E), F32)],
        interpret=INTERP,
    )(A, x2d, Wo, Wg, g2, b2)


def _route_call(gs):
    return pl.pallas_call(
        _route_body,
        in_specs=[pl.BlockSpec((T, E), lambda: (0, 0))],
        out_specs=[pl.BlockSpec((T, E), lambda: (0, 0)),
                   pl.BlockSpec((NBLK, E), lambda: (0, 0))],
        out_shape=[jax.ShapeDtypeStruct((T, E), F32),
                   jax.ShapeDtypeStruct((NBLK, E), F32)],
        scratch_shapes=[pltpu.VMEM((T, E), F32), pltpu.VMEM((T, E), F32)],
        interpret=INTERP,
    )(gs)


def _expert_call(be, Xs, W1, W2):
    nblk, npad = be.shape[0], Xs.shape[0]
    grid_spec = pltpu.PrefetchScalarGridSpec(
        num_scalar_prefetch=1,
        grid=(nblk,),
        in_specs=[pl.BlockSpec((BLK, DH), lambda b, be: (b, 0)),
                  pl.BlockSpec((1, D, CAP), lambda b, be: (be[b], 0, 0)),
                  pl.BlockSpec((1, CAP, D), lambda b, be: (be[b], 0, 0))],
        out_specs=pl.BlockSpec((BLK, DH), lambda b, be: (b, 0)),
    )
    return pl.pallas_call(
        _expert_body, grid_spec=grid_spec,
        out_shape=jax.ShapeDtypeStruct((npad, DH), F32),
        interpret=INTERP,
    )(be, Xs, W1, W2)


def _final_call(x2, G, rt):
    grid = (T // RB,)
    row = pl.BlockSpec((RB, D), lambda r: (r, 0))
    return pl.pallas_call(
        _final_body, grid=grid,
        in_specs=[row,
                  pl.BlockSpec((RB, DH), lambda r: (r, 0)),
                  pl.BlockSpec((RB, DH), lambda r: (T // RB + r, 0)),
                  pl.BlockSpec((RB, E), lambda r: (r, 0))],
        out_specs=row,
        out_shape=jax.ShapeDtypeStruct((T, D), F32),
        interpret=INTERP,
    )(x2, G, G, rt)


_SC_NC = 2
_SC_NS = 16
_NW = _SC_NC * _SC_NS


def _sc_mesh():
    return plsc.VectorSubcoreMesh(core_axis_name="c", subcore_axis_name="s",
                                  num_cores=_SC_NC, num_subcores=_SC_NS)


def _sc_wid():
    return lax.axis_index("s") * _SC_NC + lax.axis_index("c")


def _gather_ring(table_hbm, out_hbm, idx_v, rows_v, gsem, osem, base, CH, nc,
                 nbuf):
    def start_gather(i, b):
        return pltpu.async_copy(
            table_hbm.at[idx_v.at[pl.ds(i * CH, CH)]], rows_v.at[b], gsem[b])

    hg = [start_gather(i, i) for i in range(nbuf)]
    ho = [None] * nbuf
    for i in range(nc):
        b = i % nbuf
        hg[b].wait()
        ho[b] = pltpu.async_copy(
            rows_v.at[b], out_hbm.at[pl.ds(base + i * CH, CH)], osem[b])
        if i + nbuf < nc:
            ho[b].wait()
            hg[b] = start_gather(i + nbuf, b)
    for i in range(max(nc - nbuf, 0), nc):
        ho[i % nbuf].wait()


@functools.lru_cache(maxsize=None)
def _get_dispatch():
    @functools.partial(
        pl.kernel,
        out_type=jax.ShapeDtypeStruct((NP,), I32),
        mesh=_sc_mesh(),
        scratch_types=[pltpu.VMEM((NP,), I32), pltpu.VMEM((T,), I32)],
        compiler_params=pltpu.CompilerParams(needs_layout_passes=False),
    )
    def _dispatch_kernel(pos1_hbm, pos2_hbm, zeros_hbm, src_hbm, src_v, pos_v):
        @pl.when(_sc_wid() == 0)
        def _():
            pltpu.sync_copy(zeros_hbm, src_v)
            ids = lax.iota(I32, 16)

            def scat(pos_hbm):
                pltpu.sync_copy(pos_hbm, pos_v)

                def body(i, _):
                    idx = pos_v[pl.ds(i * 16, 16)]
                    plsc.store_scatter(src_v, [idx], i * 16 + ids)
                    return 0
                lax.fori_loop(0, T // 16, body, 0)

            scat(pos1_hbm)
            scat(pos2_hbm)
            pltpu.sync_copy(src_v, src_hbm)

    return _dispatch_kernel


@functools.lru_cache(maxsize=None)
def _make_gather(N, DW=DH, CH=16, nbuf=4):
    n_per = N // _NW
    nc = n_per // CH

    @functools.partial(
        pl.kernel,
        out_type=jax.ShapeDtypeStruct((N, DW), F32),
        mesh=_sc_mesh(),
        scratch_types=[pltpu.VMEM((n_per,), I32),
                       pltpu.VMEM((nbuf, CH, DW), F32)]
                      + [pltpu.SemaphoreType.DMA] * (2 * nbuf),
    )
    def gk(table_hbm, idx_hbm, out_hbm, idx_v, rows_v, *sems):
        base = _sc_wid() * n_per
        pltpu.sync_copy(idx_hbm.at[pl.ds(base, n_per)], idx_v)
        _gather_ring(table_hbm, out_hbm, idx_v, rows_v,
                     sems[:nbuf], sems[nbuf:], base, CH, nc, nbuf)

    return gk


def _dispatch_src(pos1, pos2):
    return _get_dispatch()(pos1, pos2, jnp.zeros((NP,), I32))


def _gather_rows(table, idx):
    return _make_gather(idx.shape[0])(table, idx)


def kernel(x, Wq, Wk, Wv, Wo, Wg, W1, W2, gamma1, beta1, gamma2, beta2):
    x2d = x.reshape(T, D)
    g1 = gamma1.reshape(1, D)
    b1 = beta1.reshape(1, D)
    g2 = gamma2.reshape(1, D)
    b2 = beta2.reshape(1, D)

    Q3, K3, V3 = _qkv_call(x2d, g1, b1, Wq, Wk, Wv)
    A3 = _attn_call(Q3, K3, V3)
    x2, xn2, gs = _post_call(A3, x2d, Wo, Wg, g2, b2)
    rt, be = _route_call(gs)

    pos1 = rt[:, 0].astype(I32)
    pos2 = rt[:, 1].astype(I32)
    beids = be[:, 0].astype(I32)

    src = _dispatch_src(pos1, pos2)
    Xs = _make_gather(NP, CH=8, nbuf=6)(xn2, src)
    Ys = _expert_call(beids, Xs, W1, W2)
    pos12 = jnp.concatenate([pos1, pos2])
    G = _gather_rows(Ys, pos12)
    out = _final_call(x2, G, rt)
    return out.reshape(1, T, D)
